# Initial kernel scaffold; baseline (speedup 1.0000x reference)
import jax
import jax.numpy as jnp
from jax.experimental import pallas as pl


def kernel(hidden_states, exemplar_embeddings, column_features, c_t, exemplar_aa_ids, W1, b1, W2, b2):
    raise NotImplementedError("write your pallas kernel here")


# speedup vs baseline: 2.1429x; 2.1429x over previous
import functools

import jax
import jax.numpy as jnp
from jax import lax
from jax.experimental import pallas as pl
from jax.experimental.pallas import tpu as pltpu
from jax.experimental.pallas import tpu_sc as plsc

_B, _T, _K, _L = 16, 256, 8, 1024
_H, _DE, _DF = 256, 64, 32
_V = 23
_NW = 32
_TW = (_B * _T) // _NW


def _sc_gather(ee_tab, cf_tab, aa_tab, ct_flat):
    mesh = plsc.VectorSubcoreMesh(core_axis_name="c", subcore_axis_name="s")

    @functools.partial(
        pl.kernel,
        out_type=(
            jax.ShapeDtypeStruct((_NW, _K, _TW, _DE), jnp.float32),
            jax.ShapeDtypeStruct((_NW * _TW, _DF), jnp.float32),
            jax.ShapeDtypeStruct((_NW, _TW, _K), jnp.int32),
        ),
        mesh=mesh,
        scratch_types=[
            pltpu.VMEM((_TW,), jnp.int32),
            pltpu.VMEM((_K, _TW), jnp.int32),
            pltpu.VMEM((_TW,), jnp.int32),
            pltpu.VMEM((_K, _TW, _DE), jnp.float32),
            pltpu.VMEM((_TW, _DF), jnp.float32),
            pltpu.VMEM((_TW, _K), jnp.int32),
            pltpu.SemaphoreType.DMA,
        ],
        compiler_params=pltpu.CompilerParams(use_tc_tiling_on_sc=False),
    )
    def k(ee_hbm, cf_hbm, aa_hbm, ct_hbm, ee_out, cf_out, aa_out,
          c_v, eidx_v, cidx_v, erows_v, crows_v, aarows_v, sem):
        wid = lax.axis_index("s") * 2 + lax.axis_index("c")
        b = wid // 2
        base_t = wid * _TW
        pltpu.sync_copy(ct_hbm.at[pl.ds(base_t, _TW)], c_v)
        for g in range(_TW // 16):
            c16 = c_v[pl.ds(g * 16, 16)]
            cidx_v[pl.ds(g * 16, 16)] = c16 + b * _L
            for kk in range(_K):
                eidx_v[kk, pl.ds(g * 16, 16)] = c16 + (b * _K + kk) * _L
        cps = [
            pltpu.async_copy(ee_hbm.at[eidx_v.at[kk]], erows_v.at[kk], sem)
            for kk in range(_K)
        ]
        cps.append(pltpu.async_copy(cf_hbm.at[cidx_v], crows_v, sem))
        cps.append(pltpu.async_copy(aa_hbm.at[cidx_v], aarows_v, sem))
        for cp in cps:
            cp.wait()
        pltpu.sync_copy(erows_v, ee_out.at[wid])
        pltpu.sync_copy(crows_v, cf_out.at[pl.ds(base_t, _TW)])
        pltpu.sync_copy(aarows_v, aa_out.at[wid])

    return k(ee_tab, cf_tab, aa_tab, ct_flat)


def _tc_body(hs_ref, ee_ref, cf_ref, aa_ref, w1h_ref, w1e_ref, w1f_ref,
             b1_ref, w2_ref, p_ref, lam_ref):
    hs = hs_ref[0]
    a = jnp.dot(hs, w1h_ref[...], preferred_element_type=jnp.float32)
    c = jnp.dot(cf_ref[0], w1f_ref[...], preferred_element_type=jnp.float32)
    base = a + c + b1_ref[...]
    e = jnp.dot(ee_ref[0], w1e_ref[...], preferred_element_type=jnp.float32)
    hid = jnp.maximum(e.reshape(_K, _TW, _H) + base[None], 0.0)
    scores = jnp.sum(hid * w2_ref[...][None], axis=-1)
    m = jnp.max(scores, axis=0, keepdims=True)
    ex = jnp.exp(scores - m)
    w = ex / jnp.sum(ex, axis=0, keepdims=True)
    rows = lax.broadcasted_iota(jnp.int32, (_TW, _TW), 0)
    cols = lax.broadcasted_iota(jnp.int32, (_TW, _TW), 1)
    eye = (rows == cols).astype(jnp.float32)
    wt = lax.dot_general(
        eye, w, (((1,), (1,)), ((), ())),
        preferred_element_type=jnp.float32)
    lam_ref[0] = wt
    aa = aa_ref[0]
    vv = lax.broadcasted_iota(jnp.int32, (_TW, _K, _V), 2)
    p_ref[0] = jnp.sum(
        jnp.where(aa[:, :, None] == vv, wt[:, :, None], 0.0), axis=1)


def _tc_compute(hs_r, ee_r, cf_g, aa_g, w1h, w1e, w1f, b1r, w2r):
    return pl.pallas_call(
        _tc_body,
        grid=(_NW,),
        in_specs=[
            pl.BlockSpec((1, _TW, _H), lambda i: (i, 0, 0)),
            pl.BlockSpec((1, _K * _TW, _DE), lambda i: (i, 0, 0)),
            pl.BlockSpec((1, _TW, _DF), lambda i: (i, 0, 0)),
            pl.BlockSpec((1, _TW, _K), lambda i: (i, 0, 0)),
            pl.BlockSpec((_H, _H), lambda i: (0, 0)),
            pl.BlockSpec((_DE, _H), lambda i: (0, 0)),
            pl.BlockSpec((_DF, _H), lambda i: (0, 0)),
            pl.BlockSpec((1, _H), lambda i: (0, 0)),
            pl.BlockSpec((1, _H), lambda i: (0, 0)),
        ],
        out_specs=[
            pl.BlockSpec((1, _TW, _V), lambda i: (i, 0, 0)),
            pl.BlockSpec((1, _TW, _K), lambda i: (i, 0, 0)),
        ],
        out_shape=[
            jax.ShapeDtypeStruct((_NW, _TW, _V), jnp.float32),
            jax.ShapeDtypeStruct((_NW, _TW, _K), jnp.float32),
        ],
        compiler_params=pltpu.CompilerParams(
            dimension_semantics=("parallel",)),
    )(hs_r, ee_r, cf_g, aa_g, w1h, w1e, w1f, b1r, w2r)


def kernel(hidden_states, exemplar_embeddings, column_features, c_t,
           exemplar_aa_ids, W1, b1, W2, b2):
    ee_tab = exemplar_embeddings.reshape(_B * _K * _L, _DE)
    cf_tab = column_features.reshape(_B * _L, _DF)
    aa_tab = exemplar_aa_ids.transpose(0, 2, 1).reshape(_B * _L, _K)
    ct_flat = c_t.reshape(_B * _T)

    ee_g, cf_g, aa_g = _sc_gather(ee_tab, cf_tab, aa_tab, ct_flat)

    hs_r = hidden_states.reshape(_NW, _TW, _H)
    ee_r = ee_g.reshape(_NW, _K * _TW, _DE)
    cf_r = cf_g.reshape(_NW, _TW, _DF)
    w1h = W1[:_H]
    w1e = W1[_H:_H + _DE]
    w1f = W1[_H + _DE:]
    b1r = b1.reshape(1, _H)
    w2r = W2.reshape(1, _H)

    p_blocks, lam_blocks = _tc_compute(
        hs_r, ee_r, cf_r, aa_g, w1h, w1e, w1f, b1r, w2r)
    return (p_blocks.reshape(_B, _T, _V), lam_blocks.reshape(_B, _T, _K))


# speedup vs baseline: 2.1503x; 1.0035x over previous
import functools

import jax
import jax.numpy as jnp
from jax import lax
from jax.experimental import pallas as pl
from jax.experimental.pallas import tpu as pltpu
from jax.experimental.pallas import tpu_sc as plsc

_B, _T, _K, _L = 16, 256, 8, 1024
_H, _DE, _DF = 256, 64, 32
_V = 23
_NW = 32
_TW = (_B * _T) // _NW


def _sc_gather(ee_tab, cf_tab, aa_tab, ct_flat):
    mesh = plsc.VectorSubcoreMesh(core_axis_name="c", subcore_axis_name="s")

    @functools.partial(
        pl.kernel,
        out_type=(
            jax.ShapeDtypeStruct((_NW, _K, _TW, _DE), jnp.float32),
            jax.ShapeDtypeStruct((_NW * _TW, _DF), jnp.float32),
            jax.ShapeDtypeStruct((_NW, _TW, _K), jnp.int32),
        ),
        mesh=mesh,
        scratch_types=[
            pltpu.VMEM((_TW,), jnp.int32),
            pltpu.VMEM((_K, _TW), jnp.int32),
            pltpu.VMEM((_TW,), jnp.int32),
            pltpu.VMEM((_K, _TW, _DE), jnp.float32),
            pltpu.VMEM((_TW, _DF), jnp.float32),
            pltpu.VMEM((_TW, _K), jnp.int32),
            pltpu.SemaphoreType.DMA,
        ],
        compiler_params=pltpu.CompilerParams(use_tc_tiling_on_sc=False),
    )
    def k(ee_hbm, cf_hbm, aa_hbm, ct_hbm, ee_out, cf_out, aa_out,
          c_v, eidx_v, cidx_v, erows_v, crows_v, aarows_v, sem):
        wid = lax.axis_index("s") * 2 + lax.axis_index("c")
        b = wid // 2
        base_t = wid * _TW
        pltpu.sync_copy(ct_hbm.at[pl.ds(base_t, _TW)], c_v)
        for g in range(_TW // 16):
            c16 = c_v[pl.ds(g * 16, 16)]
            cidx_v[pl.ds(g * 16, 16)] = c16 + b * _L
            for kk in range(_K):
                eidx_v[kk, pl.ds(g * 16, 16)] = c16 + (b * _K + kk) * _L
        cps = [
            pltpu.async_copy(ee_hbm.at[eidx_v.at[kk]], erows_v.at[kk], sem)
            for kk in range(_K)
        ]
        cps.append(pltpu.async_copy(aa_hbm.at[cidx_v], aarows_v, sem))
        cps.append(pltpu.async_copy(cf_hbm.at[cidx_v], crows_v, sem))
        for cp in cps:
            cp.wait()
        pltpu.sync_copy(erows_v, ee_out.at[wid])
        pltpu.sync_copy(crows_v, cf_out.at[pl.ds(base_t, _TW)])
        pltpu.sync_copy(aarows_v, aa_out.at[wid])

    return k(ee_tab, cf_tab, aa_tab, ct_flat)


def _tc_body(hs_ref, ee_ref, cf_ref, aa_ref, w1h_ref, w1e_ref, w1f_ref,
             b1_ref, w2_ref, p_ref, lam_ref):
    hs = hs_ref[0]
    a = jnp.dot(hs, w1h_ref[...], preferred_element_type=jnp.float32)
    c = jnp.dot(cf_ref[0], w1f_ref[...], preferred_element_type=jnp.float32)
    base = a + c + b1_ref[...]
    e = jnp.dot(ee_ref[0], w1e_ref[...], preferred_element_type=jnp.float32)
    hid = jnp.maximum(e.reshape(_K, _TW, _H) + base[None], 0.0)
    scores = jnp.sum(hid * w2_ref[...][None], axis=-1)
    m = jnp.max(scores, axis=0, keepdims=True)
    ex = jnp.exp(scores - m)
    w = ex / jnp.sum(ex, axis=0, keepdims=True)
    rows = lax.broadcasted_iota(jnp.int32, (_TW, _TW), 0)
    cols = lax.broadcasted_iota(jnp.int32, (_TW, _TW), 1)
    eye = (rows == cols).astype(jnp.float32)
    lam = lax.dot_general(
        eye, w, (((1,), (1,)), ((), ())),
        preferred_element_type=jnp.float32)
    lam_ref[0] = lam
    aa = aa_ref[0]
    vv = lax.broadcasted_iota(jnp.int32, (_TW, _K, _V), 2)
    p_ref[0] = jnp.sum(
        jnp.where(aa[:, :, None] == vv, lam[:, :, None], 0.0), axis=1)


def _tc_compute(hs_r, ee_r, cf_g, aa_g, w1h, w1e, w1f, b1r, w2r):
    return pl.pallas_call(
        _tc_body,
        grid=(_NW,),
        in_specs=[
            pl.BlockSpec((1, _TW, _H), lambda i: (i, 0, 0)),
            pl.BlockSpec((1, _K * _TW, _DE), lambda i: (i, 0, 0)),
            pl.BlockSpec((1, _TW, _DF), lambda i: (i, 0, 0)),
            pl.BlockSpec((1, _TW, _K), lambda i: (i, 0, 0)),
            pl.BlockSpec((_H, _H), lambda i: (0, 0)),
            pl.BlockSpec((_DE, _H), lambda i: (0, 0)),
            pl.BlockSpec((_DF, _H), lambda i: (0, 0)),
            pl.BlockSpec((1, _H), lambda i: (0, 0)),
            pl.BlockSpec((1, _H), lambda i: (0, 0)),
        ],
        out_specs=[
            pl.BlockSpec((1, _TW, _V), lambda i: (i, 0, 0)),
            pl.BlockSpec((1, _TW, _K), lambda i: (i, 0, 0)),
        ],
        out_shape=[
            jax.ShapeDtypeStruct((_NW, _TW, _V), jnp.float32),
            jax.ShapeDtypeStruct((_NW, _TW, _K), jnp.float32),
        ],
        compiler_params=pltpu.CompilerParams(
            dimension_semantics=("parallel",)),
    )(hs_r, ee_r, cf_g, aa_g, w1h, w1e, w1f, b1r, w2r)


def kernel(hidden_states, exemplar_embeddings, column_features, c_t,
           exemplar_aa_ids, W1, b1, W2, b2):
    ee_tab = exemplar_embeddings.reshape(_B * _K * _L, _DE)
    cf_tab = column_features.reshape(_B * _L, _DF)
    aa_tab = exemplar_aa_ids.transpose(0, 2, 1).reshape(_B * _L, _K)
    ct_flat = c_t.reshape(_B * _T)

    ee_g, cf_g, aa_g = _sc_gather(ee_tab, cf_tab, aa_tab, ct_flat)

    hs_r = hidden_states.reshape(_NW, _TW, _H)
    ee_r = ee_g.reshape(_NW, _K * _TW, _DE)
    cf_r = cf_g.reshape(_NW, _TW, _DF)
    w1h = W1[:_H]
    w1e = W1[_H:_H + _DE]
    w1f = W1[_H + _DE:]
    b1r = b1.reshape(1, _H)
    w2r = W2.reshape(1, _H)

    p_blocks, lam_blocks = _tc_compute(
        hs_r, ee_r, cf_r, aa_g, w1h, w1e, w1f, b1r, w2r)
    return (p_blocks.reshape(_B, _T, _V), lam_blocks.reshape(_B, _T, _K))
